# baseline (speedup 1.0000x reference)
import functools

import jax
import jax.numpy as jnp
from jax import lax
from jax.experimental import pallas as pl
from jax.experimental.pallas import tpu as pltpu
from jax.experimental.pallas import tpu_sc as plsc

_BN_EPS = 1e-5
_NC = 2
_NS = 16
_K = 128
_B = 16


def _make_agg(n_pad, d, n_chunks, fsplit):
    r_rows = n_pad // _NS
    n_blocks = n_chunks // _B
    tab_rows = (_NC if fsplit else 1) * n_pad

    @functools.partial(
        pl.kernel,
        out_type=jax.ShapeDtypeStruct((_NC * n_pad, d), jnp.float32),
        mesh=plsc.VectorSubcoreMesh(core_axis_name="c", subcore_axis_name="s"),
        scratch_types=[
            pltpu.VMEM((_B, _K), jnp.int32),
            pltpu.VMEM((_B, _K), jnp.int32),
            pltpu.VMEM((_K, d), jnp.float32),
            pltpu.VMEM((_K, d), jnp.float32),
            pltpu.SemaphoreType.DMA,
            pltpu.SemaphoreType.DMA,
            pltpu.SemaphoreType.DMA,
            pltpu.SemaphoreType.DMA,
            pltpu.VMEM_SHARED((n_pad, d), jnp.float32),
        ],
    )
    def agg(tab_hbm, src_hbm, dst_hbm, out_hbm, src_v, dst_v, rows0, rows1,
            sem_g0, sem_g1, sem_s0, sem_s1, acc_sh):
        c = lax.axis_index("c")
        s = lax.axis_index("s")
        cw = c if fsplit else 1 - c
        src_base = (cw * _NS + s) * n_chunks
        dst_base = (s if fsplit else cw * _NS + s) * n_chunks
        init_off = c * n_pad if fsplit else 0
        pltpu.sync_copy(tab_hbm.at[pl.ds(init_off + s * r_rows, r_rows)],
                        acc_sh.at[pl.ds(s * r_rows, r_rows)])
        plsc.subcore_barrier()

        @pl.loop(0, n_blocks)
        def _(b):
            pltpu.sync_copy(src_hbm.at[pl.ds(src_base + b * _B, _B)], src_v)
            pltpu.sync_copy(dst_hbm.at[pl.ds(dst_base + b * _B, _B)], dst_v)
            pltpu.async_copy(tab_hbm.at[src_v.at[0]], rows0, sem_g0)
            pltpu.async_copy(tab_hbm.at[src_v.at[1]], rows1, sem_g1)

            @pl.loop(0, _B // 2)
            def _(k):
                j0 = 2 * k
                for rows, sem_g, sem_s, j in ((rows0, sem_g0, sem_s0, j0),
                                              (rows1, sem_g1, sem_s1, j0 + 1)):
                    pltpu.make_async_copy(tab_hbm.at[src_v.at[j]], rows,
                                          sem_g).wait()
                    pltpu.async_copy(rows, acc_sh.at[dst_v.at[j]], sem_s,
                                     add=True)
                for rows, sem_g, sem_s, j in ((rows0, sem_g0, sem_s0, j0),
                                              (rows1, sem_g1, sem_s1, j0 + 1)):
                    pltpu.make_async_copy(rows, acc_sh.at[dst_v.at[j]],
                                          sem_s).wait()

                    @pl.when(k < _B // 2 - 1)
                    def _():
                        pltpu.async_copy(tab_hbm.at[src_v.at[j + 2]], rows,
                                         sem_g)

        plsc.subcore_barrier()
        pltpu.sync_copy(acc_sh.at[pl.ds(s * r_rows, r_rows)],
                        out_hbm.at[pl.ds(c * n_pad + s * r_rows, r_rows)])

    return agg


def _tc1_body(n, n_pad, dh, p_ref, x_ref, w1_ref, b1_ref, w2_ref, b2_ref,
              g1_ref, beta1_ref, out_ref):
    a = p_ref[:n] + p_ref[n_pad:n_pad + n] - x_ref[...]
    h = jnp.dot(a, w1_ref[...], preferred_element_type=jnp.float32)
    h = jnp.maximum(h + b1_ref[...], 0.0)
    h = jnp.dot(h, w2_ref[...], preferred_element_type=jnp.float32) + b2_ref[...]
    m = jnp.mean(h, axis=0, keepdims=True)
    v = jnp.mean((h - m) * (h - m), axis=0, keepdims=True)
    h = (h - m) * lax.rsqrt(v + _BN_EPS) * g1_ref[...] + beta1_ref[...]
    h = jnp.maximum(h, 0.0)
    dhalf = dh // 2
    zpad = jnp.zeros((n_pad - n, dhalf), jnp.float32)
    out_ref[:n] = h[:, :dhalf]
    out_ref[n:n_pad] = zpad
    out_ref[n_pad:n_pad + n] = h[:, dhalf:]
    out_ref[n_pad + n:] = zpad


def _tc2_body(n, n_pad, g, p_ref, w3_ref, b3_ref, g2_ref, beta2_ref, bt_ref,
              out_ref):
    dhalf = w3_ref.shape[0] // 2
    z = jnp.dot(p_ref[:n], w3_ref[:dhalf], preferred_element_type=jnp.float32)
    z = z + jnp.dot(p_ref[n_pad:n_pad + n], w3_ref[dhalf:],
                    preferred_element_type=jnp.float32)
    z = z + b3_ref[...]
    m = jnp.mean(z, axis=0, keepdims=True)
    v = jnp.mean((z - m) * (z - m), axis=0, keepdims=True)
    h2 = (z - m) * lax.rsqrt(v + _BN_EPS) * g2_ref[...] + beta2_ref[...]
    h2 = jnp.maximum(h2, 0.0)
    gid = lax.broadcasted_iota(jnp.int32, (g, n), 0)
    mask = (jnp.broadcast_to(bt_ref[...], (g, n)) == gid).astype(jnp.float32)
    out_ref[...] = jnp.dot(mask, h2, preferred_element_type=jnp.float32)


def kernel(x, edge_index, batch, W1, b1, W2, b2, W3, b3, g1, beta1, g2, beta2):
    if edge_index.ndim > 2:
        edge_index = edge_index.reshape(2, -1)
    src = edge_index[0]
    dst = edge_index[1]
    n, din = x.shape
    dh = W1.shape[1]
    dout = W3.shape[1]
    g = 64
    e = src.shape[0]

    n_pad = ((n // 128) + 1) * 128
    c1 = -(-e // (_NC * _NS * _K * _B)) * _B
    c2 = 2 * c1
    e_pad = _NC * _NS * _K * c1
    pad = e_pad - e
    src_p = jnp.concatenate([src, jnp.zeros((pad,), jnp.int32)])
    dummy_dst = n + jnp.arange(pad, dtype=jnp.int32) % (n_pad - n)
    dst_p = jnp.concatenate([dst, dummy_dst])
    src1 = src_p.reshape(-1, _K)
    dst1 = dst_p.reshape(-1, _K)
    offs = (jnp.arange(_NC, dtype=jnp.int32) * n_pad)[:, None]
    src2 = (src_p[None] + offs).reshape(-1, _K)
    x_pad = jnp.concatenate([x, jnp.zeros((n_pad - n, din), jnp.float32)])

    p = _make_agg(n_pad, din, c1, fsplit=False)(x_pad, src1, dst1)

    hstack = pl.pallas_call(
        functools.partial(_tc1_body, n, n_pad, dh),
        out_shape=jax.ShapeDtypeStruct((2 * n_pad, dh // 2), jnp.float32),
    )(p, x, W1, b1.reshape(1, dh), W2, b2.reshape(1, dh),
      g1.reshape(1, dh), beta1.reshape(1, dh))

    q = _make_agg(n_pad, dh // 2, c2, fsplit=True)(hstack, src2, dst1)

    out = pl.pallas_call(
        functools.partial(_tc2_body, n, n_pad, g),
        out_shape=jax.ShapeDtypeStruct((g, dout), jnp.float32),
    )(q, W3, b3.reshape(1, dout), g2.reshape(1, dout),
      beta2.reshape(1, dout), batch.reshape(1, n))
    return out


# speedup vs baseline: 2.5173x; 2.5173x over previous
import functools

import jax
import jax.numpy as jnp
from jax import lax
from jax.experimental import pallas as pl
from jax.experimental.pallas import tpu as pltpu
from jax.experimental.pallas import tpu_sc as plsc

_BN_EPS = 1e-5
_NC = 2
_NS = 16
_K = 128
_B = 16


def _make_agg(n_pad, d, n_chunks, fsplit):
    r_rows = n_pad // _NS
    n_blocks = n_chunks // _B
    tab_rows = (_NC if fsplit else 1) * n_pad

    @functools.partial(
        pl.kernel,
        out_type=jax.ShapeDtypeStruct((_NC * n_pad, d), jnp.float32),
        mesh=plsc.VectorSubcoreMesh(core_axis_name="c", subcore_axis_name="s"),
        scratch_types=[
            pltpu.VMEM((_B, _K), jnp.int32),
            pltpu.VMEM((_B, _K), jnp.int32),
            pltpu.VMEM((_K, d), jnp.float32),
            pltpu.VMEM((_K, d), jnp.float32),
            pltpu.SemaphoreType.DMA,
            pltpu.SemaphoreType.DMA,
            pltpu.SemaphoreType.DMA,
            pltpu.SemaphoreType.DMA,
            pltpu.VMEM_SHARED((n_pad, d), jnp.float32),
        ],
    )
    def agg(tab_hbm, src_hbm, dst_hbm, out_hbm, src_v, dst_v, rows0, rows1,
            sem_g0, sem_g1, sem_s0, sem_s1, acc_sh):
        c = lax.axis_index("c")
        s = lax.axis_index("s")
        src_base = (c * _NS + s) * n_chunks
        dst_base = (s if fsplit else c * _NS + s) * n_chunks
        init_off = c * n_pad if fsplit else 0
        pltpu.sync_copy(tab_hbm.at[pl.ds(init_off + s * r_rows, r_rows)],
                        acc_sh.at[pl.ds(s * r_rows, r_rows)])
        plsc.subcore_barrier()

        @pl.loop(0, n_blocks)
        def _(b):
            pltpu.sync_copy(src_hbm.at[pl.ds(src_base + b * _B, _B)], src_v)
            pltpu.sync_copy(dst_hbm.at[pl.ds(dst_base + b * _B, _B)], dst_v)
            pltpu.async_copy(tab_hbm.at[src_v.at[0]], rows0, sem_g0)
            pltpu.async_copy(tab_hbm.at[src_v.at[1]], rows1, sem_g1)

            @pl.loop(0, _B // 2)
            def _(k):
                j0 = 2 * k
                for rows, sem_g, sem_s, j in ((rows0, sem_g0, sem_s0, j0),
                                              (rows1, sem_g1, sem_s1, j0 + 1)):
                    pltpu.make_async_copy(tab_hbm.at[src_v.at[j]], rows,
                                          sem_g).wait()
                    pltpu.async_copy(rows, acc_sh.at[dst_v.at[j]], sem_s,
                                     add=True)
                for rows, sem_g, sem_s, j in ((rows0, sem_g0, sem_s0, j0),
                                              (rows1, sem_g1, sem_s1, j0 + 1)):
                    pltpu.make_async_copy(rows, acc_sh.at[dst_v.at[j]],
                                          sem_s).wait()

                    @pl.when(k < _B // 2 - 1)
                    def _():
                        pltpu.async_copy(tab_hbm.at[src_v.at[j + 2]], rows,
                                         sem_g)

        plsc.subcore_barrier()
        pltpu.sync_copy(acc_sh.at[pl.ds(s * r_rows, r_rows)],
                        out_hbm.at[pl.ds(c * n_pad + s * r_rows, r_rows)])

    return agg


def _tc1_body(n, n_pad, dh, p_ref, x_ref, w1_ref, b1_ref, w2_ref, b2_ref,
              g1_ref, beta1_ref, out_ref):
    a = p_ref[:n] + p_ref[n_pad:n_pad + n] - x_ref[...]
    h = jnp.dot(a, w1_ref[...], preferred_element_type=jnp.float32)
    h = jnp.maximum(h + b1_ref[...], 0.0)
    h = jnp.dot(h, w2_ref[...], preferred_element_type=jnp.float32) + b2_ref[...]
    m = jnp.mean(h, axis=0, keepdims=True)
    v = jnp.mean((h - m) * (h - m), axis=0, keepdims=True)
    h = (h - m) * lax.rsqrt(v + _BN_EPS) * g1_ref[...] + beta1_ref[...]
    h = jnp.maximum(h, 0.0)
    dhalf = dh // 2
    zpad = jnp.zeros((n_pad - n, dhalf), jnp.float32)
    out_ref[:n] = h[:, :dhalf]
    out_ref[n:n_pad] = zpad
    out_ref[n_pad:n_pad + n] = h[:, dhalf:]
    out_ref[n_pad + n:] = zpad


def _tc2_body(n, n_pad, g, p_ref, w3_ref, b3_ref, g2_ref, beta2_ref, bt_ref,
              out_ref):
    dhalf = w3_ref.shape[0] // 2
    z = jnp.dot(p_ref[:n], w3_ref[:dhalf], preferred_element_type=jnp.float32)
    z = z + jnp.dot(p_ref[n_pad:n_pad + n], w3_ref[dhalf:],
                    preferred_element_type=jnp.float32)
    z = z + b3_ref[...]
    m = jnp.mean(z, axis=0, keepdims=True)
    v = jnp.mean((z - m) * (z - m), axis=0, keepdims=True)
    h2 = (z - m) * lax.rsqrt(v + _BN_EPS) * g2_ref[...] + beta2_ref[...]
    h2 = jnp.maximum(h2, 0.0)
    gid = lax.broadcasted_iota(jnp.int32, (g, n), 0)
    mask = (jnp.broadcast_to(bt_ref[...], (g, n)) == gid).astype(jnp.float32)
    out_ref[...] = jnp.dot(mask, h2, preferred_element_type=jnp.float32)


def kernel(x, edge_index, batch, W1, b1, W2, b2, W3, b3, g1, beta1, g2, beta2):
    if edge_index.ndim > 2:
        edge_index = edge_index.reshape(2, -1)
    src = edge_index[0]
    dst = edge_index[1]
    n, din = x.shape
    dh = W1.shape[1]
    dout = W3.shape[1]
    g = 64
    e = src.shape[0]

    n_pad = ((n // 128) + 1) * 128
    c1 = -(-e // (_NC * _NS * _K * _B)) * _B
    c2 = 2 * c1
    e_pad = _NC * _NS * _K * c1
    pad = e_pad - e
    iar = jnp.arange(pad, dtype=jnp.int32)
    src_p = jnp.concatenate([src, iar % n])
    dst_p = jnp.concatenate([dst, n + iar % (n_pad - n)])
    src1 = src_p.reshape(-1, _K)
    dst1 = dst_p.reshape(-1, _K)
    offs = (jnp.arange(_NC, dtype=jnp.int32) * n_pad)[:, None]
    src2 = (src_p[None] + offs).reshape(-1, _K)
    x_pad = jnp.concatenate([x, jnp.zeros((n_pad - n, din), jnp.float32)])

    p = _make_agg(n_pad, din, c1, fsplit=False)(x_pad, src1, dst1)

    hstack = pl.pallas_call(
        functools.partial(_tc1_body, n, n_pad, dh),
        out_shape=jax.ShapeDtypeStruct((2 * n_pad, dh // 2), jnp.float32),
    )(p, x, W1, b1.reshape(1, dh), W2, b2.reshape(1, dh),
      g1.reshape(1, dh), beta1.reshape(1, dh))

    q = _make_agg(n_pad, dh // 2, c2, fsplit=True)(hstack, src2, dst1)

    out = pl.pallas_call(
        functools.partial(_tc2_body, n, n_pad, g),
        out_shape=jax.ShapeDtypeStruct((g, dout), jnp.float32),
    )(q, W3, b3.reshape(1, dout), g2.reshape(1, dout),
      beta2.reshape(1, dout), batch.reshape(1, n))
    return out


# speedup vs baseline: 3.4590x; 1.3741x over previous
import functools

import jax
import jax.numpy as jnp
from jax import lax
from jax.experimental import pallas as pl
from jax.experimental.pallas import tpu as pltpu
from jax.experimental.pallas import tpu_sc as plsc

_BN_EPS = 1e-5
_NC = 2
_NS = 16
_K = 128
_B = 16


def _make_agg(n_pad, d, n_chunks, fsplit):
    r_rows = n_pad // _NS
    n_blocks = n_chunks // _B
    tab_rows = (_NC if fsplit else 1) * n_pad

    @functools.partial(
        pl.kernel,
        out_type=jax.ShapeDtypeStruct((_NC * n_pad, d), jnp.float32),
        mesh=plsc.VectorSubcoreMesh(core_axis_name="c", subcore_axis_name="s"),
        scratch_types=[
            pltpu.VMEM((_B, _K), jnp.int32),
            pltpu.VMEM((_B, _K), jnp.int32),
            pltpu.VMEM((_K, d), jnp.float32),
            pltpu.VMEM((_K, d), jnp.float32),
            pltpu.SemaphoreType.DMA,
            pltpu.SemaphoreType.DMA,
            pltpu.SemaphoreType.DMA,
            pltpu.SemaphoreType.DMA,
            pltpu.VMEM_SHARED((n_pad, d), jnp.float32),
        ],
    )
    def agg(tab_hbm, src_hbm, dst_hbm, out_hbm, src_v, dst_v, rows0, rows1,
            sem_g0, sem_g1, sem_s0, sem_s1, acc_sh):
        c = lax.axis_index("c")
        s = lax.axis_index("s")
        src_base = (c * _NS + s) * n_chunks
        dst_base = (s if fsplit else c * _NS + s) * n_chunks
        init_off = c * n_pad if fsplit else 0
        pltpu.sync_copy(tab_hbm.at[pl.ds(init_off + s * r_rows, r_rows)],
                        acc_sh.at[pl.ds(s * r_rows, r_rows)])
        plsc.subcore_barrier()

        @pl.loop(0, n_blocks)
        def _(b):
            pltpu.sync_copy(src_hbm.at[pl.ds(src_base + b * _B, _B)], src_v)
            pltpu.sync_copy(dst_hbm.at[pl.ds(dst_base + b * _B, _B)], dst_v)
            pltpu.async_copy(tab_hbm.at[src_v.at[0]], rows0, sem_g0)
            pltpu.async_copy(tab_hbm.at[src_v.at[1]], rows1, sem_g1)

            @pl.loop(0, _B // 2)
            def _(k):
                j0 = 2 * k
                for rows, sem_g, sem_s, j in ((rows0, sem_g0, sem_s0, j0),
                                              (rows1, sem_g1, sem_s1, j0 + 1)):
                    pltpu.make_async_copy(tab_hbm.at[src_v.at[j]], rows,
                                          sem_g).wait()
                    pltpu.async_copy(rows, acc_sh.at[dst_v.at[j]], sem_s,
                                     add=True)
                for rows, sem_g, sem_s, j in ((rows0, sem_g0, sem_s0, j0),
                                              (rows1, sem_g1, sem_s1, j0 + 1)):
                    pltpu.make_async_copy(rows, acc_sh.at[dst_v.at[j]],
                                          sem_s).wait()

                    @pl.when(k < _B // 2 - 1)
                    def _():
                        pltpu.async_copy(tab_hbm.at[src_v.at[j + 2]], rows,
                                         sem_g)

        plsc.subcore_barrier()
        pltpu.sync_copy(acc_sh.at[pl.ds(s * r_rows, r_rows)],
                        out_hbm.at[pl.ds(c * n_pad + s * r_rows, r_rows)])

    return agg


def _tc1_body(n, n_pad, p_ref, x_ref, w1_ref, b1_ref, w2_ref, b2_ref,
              g1_ref, beta1_ref, w3_ref, out_ref):
    a = p_ref[:n] + p_ref[n_pad:n_pad + n] - x_ref[...]
    h = jnp.dot(a, w1_ref[...], preferred_element_type=jnp.float32)
    h = jnp.maximum(h + b1_ref[...], 0.0)
    h = jnp.dot(h, w2_ref[...], preferred_element_type=jnp.float32) + b2_ref[...]
    m = jnp.mean(h, axis=0, keepdims=True)
    v = jnp.mean((h - m) * (h - m), axis=0, keepdims=True)
    h = (h - m) * lax.rsqrt(v + _BN_EPS) * g1_ref[...] + beta1_ref[...]
    h = jnp.maximum(h, 0.0)
    z = jnp.dot(h, w3_ref[...], preferred_element_type=jnp.float32)
    out_ref[:n] = z
    out_ref[n:] = jnp.zeros((n_pad - n, z.shape[1]), jnp.float32)


def _tc2_body(n, n_pad, g, q_ref, z_ref, b3_ref, g2_ref, beta2_ref, bt_ref,
              out_ref):
    z = q_ref[:n] + q_ref[n_pad:n_pad + n] - z_ref[:n] + b3_ref[...]
    m = jnp.mean(z, axis=0, keepdims=True)
    v = jnp.mean((z - m) * (z - m), axis=0, keepdims=True)
    h2 = (z - m) * lax.rsqrt(v + _BN_EPS) * g2_ref[...] + beta2_ref[...]
    h2 = jnp.maximum(h2, 0.0)
    gid = lax.broadcasted_iota(jnp.int32, (g, n), 0)
    mask = (jnp.broadcast_to(bt_ref[...], (g, n)) == gid).astype(jnp.float32)
    out_ref[...] = jnp.dot(mask, h2, preferred_element_type=jnp.float32)


def kernel(x, edge_index, batch, W1, b1, W2, b2, W3, b3, g1, beta1, g2, beta2):
    if edge_index.ndim > 2:
        edge_index = edge_index.reshape(2, -1)
    src = edge_index[0]
    dst = edge_index[1]
    n, din = x.shape
    dh = W1.shape[1]
    dout = W3.shape[1]
    g = 64
    e = src.shape[0]

    n_pad = ((n // 128) + 1) * 128
    c1 = -(-e // (_NC * _NS * _K * _B)) * _B
    e_pad = _NC * _NS * _K * c1
    pad = e_pad - e
    iar = jnp.arange(pad, dtype=jnp.int32)
    src_p = jnp.concatenate([src, iar % n])
    dst_p = jnp.concatenate([dst, n + iar % (n_pad - n)])
    src1 = src_p.reshape(-1, _K)
    dst1 = dst_p.reshape(-1, _K)
    x_pad = jnp.concatenate([x, jnp.zeros((n_pad - n, din), jnp.float32)])

    agg = _make_agg(n_pad, din, c1, fsplit=False)

    p = agg(x_pad, src1, dst1)

    z = pl.pallas_call(
        functools.partial(_tc1_body, n, n_pad),
        out_shape=jax.ShapeDtypeStruct((n_pad, dout), jnp.float32),
    )(p, x, W1, b1.reshape(1, dh), W2, b2.reshape(1, dh),
      g1.reshape(1, dh), beta1.reshape(1, dh), W3)

    q = agg(z, src1, dst1)

    out = pl.pallas_call(
        functools.partial(_tc2_body, n, n_pad, g),
        out_shape=jax.ShapeDtypeStruct((g, dout), jnp.float32),
    )(q, z, b3.reshape(1, dout), g2.reshape(1, dout),
      beta2.reshape(1, dout), batch.reshape(1, n))
    return out


# speedup vs baseline: 3.6362x; 1.0512x over previous
import functools

import jax
import jax.numpy as jnp
from jax import lax
from jax.experimental import pallas as pl
from jax.experimental.pallas import tpu as pltpu
from jax.experimental.pallas import tpu_sc as plsc

_BN_EPS = 1e-5
_NC = 2
_NS = 16
_K = 128
_B = 8


def _make_agg(n_pad, d, n_chunks):
    r_rows = n_pad // _NS
    n_blocks = n_chunks // _B
    assert n_blocks % 2 == 0

    @functools.partial(
        pl.kernel,
        out_type=jax.ShapeDtypeStruct((_NC * n_pad, d), jnp.float32),
        mesh=plsc.VectorSubcoreMesh(core_axis_name="c", subcore_axis_name="s"),
        scratch_types=[
            pltpu.VMEM((2 * _B, _K), jnp.int32),
            pltpu.VMEM((2 * _B, _K), jnp.int32),
            pltpu.VMEM((_K, d), jnp.float32),
            pltpu.VMEM((_K, d), jnp.float32),
            pltpu.SemaphoreType.DMA,
            pltpu.SemaphoreType.DMA,
            pltpu.SemaphoreType.DMA,
            pltpu.SemaphoreType.DMA,
            pltpu.SemaphoreType.DMA,
            pltpu.SemaphoreType.DMA,
            pltpu.VMEM_SHARED((n_pad, d), jnp.float32),
        ],
    )
    def agg(tab_hbm, idx_hbm, out_hbm, i0, i1, rows0, rows1,
            sem_i0, sem_i1, sem_g0, sem_g1, sem_s0, sem_s1, acc_sh):
        c = lax.axis_index("c")
        s = lax.axis_index("s")
        base = (c * _NS + s) * n_blocks

        def idx_block(b, ibuf, sem):
            return pltpu.make_async_copy(
                idx_hbm.at[pl.ds((base + b) * 2 * _B, 2 * _B)], ibuf, sem)

        def gather(j, rows, sem):
            ibuf = (i0, i1)[j // _B]
            return pltpu.make_async_copy(tab_hbm.at[ibuf.at[j % _B]], rows,
                                         sem)

        def scat_start(j, rows, sem):
            ibuf = (i0, i1)[j // _B]
            pltpu.async_copy(rows, acc_sh.at[ibuf.at[_B + j % _B]], sem,
                             add=True)

        def scat_wait(j, rows, sem):
            ibuf = (i0, i1)[j // _B]
            pltpu.make_async_copy(rows, acc_sh.at[ibuf.at[_B + j % _B]],
                                  sem).wait()

        idx_block(0, i0, sem_i0).start()
        idx_block(1, i1, sem_i1).start()
        idx_block(0, i0, sem_i0).wait()
        gather(0, rows0, sem_g0).start()
        gather(1, rows1, sem_g1).start()
        pltpu.sync_copy(tab_hbm.at[pl.ds(s * r_rows, r_rows)],
                        acc_sh.at[pl.ds(s * r_rows, r_rows)])
        plsc.subcore_barrier()

        @pl.loop(0, n_blocks, step=2)
        def _(b):
            for k in range(_B):
                j0, j1 = 2 * k, 2 * k + 1
                gather(j0, rows0, sem_g0).wait()
                scat_start(j0, rows0, sem_s0)
                gather(j1, rows1, sem_g1).wait()
                scat_start(j1, rows1, sem_s1)
                if j0 + 2 == _B:
                    idx_block(0, i1, sem_i1).wait()
                scat_wait(j0, rows0, sem_s0)
                if j0 + 2 < 2 * _B:
                    gather(j0 + 2, rows0, sem_g0).start()
                else:

                    @pl.when(b + 2 < n_blocks)
                    def _():
                        idx_block(0, i0, sem_i0).wait()
                        gather(0, rows0, sem_g0).start()
                scat_wait(j1, rows1, sem_s1)
                if j1 + 2 < 2 * _B:
                    gather(j1 + 2, rows1, sem_g1).start()
                else:

                    @pl.when(b + 2 < n_blocks)
                    def _():
                        gather(1, rows1, sem_g1).start()
                if j1 == _B - 1:
                    @pl.when(b + 2 < n_blocks)
                    def _():
                        idx_block(b + 2, i0, sem_i0).start()

            @pl.when(b + 3 < n_blocks)
            def _():
                idx_block(b + 3, i1, sem_i1).start()

        plsc.subcore_barrier()
        pltpu.sync_copy(acc_sh.at[pl.ds(s * r_rows, r_rows)],
                        out_hbm.at[pl.ds(c * n_pad + s * r_rows, r_rows)])

    return agg


def _tc1_body(n, n_pad, p_ref, x_ref, w1_ref, b1_ref, w2_ref, b2_ref,
              g1_ref, beta1_ref, w3_ref, out_ref):
    a = p_ref[:n] + p_ref[n_pad:n_pad + n] - x_ref[...]
    h = jnp.dot(a, w1_ref[...], preferred_element_type=jnp.float32)
    h = jnp.maximum(h + b1_ref[...], 0.0)
    h = jnp.dot(h, w2_ref[...], preferred_element_type=jnp.float32) + b2_ref[...]
    m = jnp.mean(h, axis=0, keepdims=True)
    v = jnp.mean((h - m) * (h - m), axis=0, keepdims=True)
    h = (h - m) * lax.rsqrt(v + _BN_EPS) * g1_ref[...] + beta1_ref[...]
    h = jnp.maximum(h, 0.0)
    z = jnp.dot(h, w3_ref[...], preferred_element_type=jnp.float32)
    out_ref[:n] = z
    out_ref[n:] = jnp.zeros((n_pad - n, z.shape[1]), jnp.float32)


def _tc2_body(n, n_pad, g, q_ref, z_ref, b3_ref, g2_ref, beta2_ref, bt_ref,
              out_ref):
    z = q_ref[:n] + q_ref[n_pad:n_pad + n] - z_ref[:n] + b3_ref[...]
    m = jnp.mean(z, axis=0, keepdims=True)
    v = jnp.mean((z - m) * (z - m), axis=0, keepdims=True)
    h2 = (z - m) * lax.rsqrt(v + _BN_EPS) * g2_ref[...] + beta2_ref[...]
    h2 = jnp.maximum(h2, 0.0)
    gid = lax.broadcasted_iota(jnp.int32, (g, n), 0)
    mask = (jnp.broadcast_to(bt_ref[...], (g, n)) == gid).astype(jnp.float32)
    out_ref[...] = jnp.dot(mask, h2, preferred_element_type=jnp.float32)


def kernel(x, edge_index, batch, W1, b1, W2, b2, W3, b3, g1, beta1, g2, beta2):
    if edge_index.ndim > 2:
        edge_index = edge_index.reshape(2, -1)
    src = edge_index[0]
    dst = edge_index[1]
    n, din = x.shape
    dh = W1.shape[1]
    dout = W3.shape[1]
    g = 64
    e = src.shape[0]

    n_pad = ((n // 128) + 1) * 128
    nb = -(-e // (_NC * _NS * _K * _B))
    nb += nb % 2
    c1 = nb * _B
    e_pad = _NC * _NS * _K * c1
    pad = e_pad - e
    iar = jnp.arange(pad, dtype=jnp.int32)
    src_p = jnp.concatenate([src, iar % n])
    dst_p = jnp.concatenate([dst, n + iar % (n_pad - n)])
    nw = _NC * _NS
    n_blocks = c1 // _B
    src_b = src_p.reshape(nw, n_blocks, _B, _K)
    dst_b = dst_p.reshape(nw, n_blocks, _B, _K)
    idx1 = jnp.concatenate([src_b, dst_b], axis=2).reshape(-1, _K)
    x_pad = jnp.concatenate([x, jnp.zeros((n_pad - n, din), jnp.float32)])

    agg = _make_agg(n_pad, din, c1)

    p = agg(x_pad, idx1)

    z = pl.pallas_call(
        functools.partial(_tc1_body, n, n_pad),
        out_shape=jax.ShapeDtypeStruct((n_pad, dout), jnp.float32),
    )(p, x, W1, b1.reshape(1, dh), W2, b2.reshape(1, dh),
      g1.reshape(1, dh), beta1.reshape(1, dh), W3)

    q = agg(z, idx1)

    out = pl.pallas_call(
        functools.partial(_tc2_body, n, n_pad, g),
        out_shape=jax.ShapeDtypeStruct((g, dout), jnp.float32),
    )(q, z, b3.reshape(1, dout), g2.reshape(1, dout),
      beta2.reshape(1, dout), batch.reshape(1, n))
    return out
